# baseline (speedup 1.0000x reference)
import functools

import jax
import jax.numpy as jnp
from jax import lax
from jax.experimental import pallas as pl
from jax.experimental.pallas import tpu as pltpu
from jax.experimental.pallas import tpu_sc as plsc

M = 100000
D = 128
B = 16384
NW = 32
BPW = B // NW
IC = 128
NB_G = BPW // IC
SH = 3136
SH_LAST = M - (NW - 1) * SH
NV = SH // 16
SELR = SH // IC + 1
CP = 112
NCP = SH // CP
NCP_LAST = SH_LAST // CP
CP_TAIL = SH_LAST - NCP_LAST * CP

_mesh = plsc.VectorSubcoreMesh(core_axis_name="c", subcore_axis_name="s")
_sc_params = pltpu.CompilerParams(needs_layout_passes=False)


@functools.partial(
    pl.kernel,
    out_type=jax.ShapeDtypeStruct((B, D), jnp.float32),
    mesh=_mesh,
    scratch_types=[
        pltpu.VMEM((BPW,), jnp.int32),
        pltpu.VMEM((BPW, D), jnp.float32),
        pltpu.SemaphoreType.DMA,
    ],
    compiler_params=_sc_params,
)
def _sc_gather(mem_hbm, ids_hbm, h_hbm, idx_v, rows_v, sem):
    wid = lax.axis_index("s") * 2 + lax.axis_index("c")
    base = wid * BPW
    pltpu.sync_copy(ids_hbm.at[pl.ds(base, BPW)], idx_v)
    for k in range(NB_G):
        pltpu.async_copy(mem_hbm.at[idx_v.at[pl.ds(k * IC, IC)]],
                         rows_v.at[pl.ds(k * IC, IC)], sem)
    for k in range(NB_G):
        pltpu.make_async_copy(mem_hbm.at[idx_v.at[pl.ds(k * IC, IC)]],
                              rows_v.at[pl.ds(k * IC, IC)], sem).wait()
    pltpu.sync_copy(rows_v, h_hbm.at[pl.ds(base, BPW)])


def _gru_block(x_ref, h_ref, wih_ref, whh_ref, bih_ref, bhh_ref, out_ref):
    x = x_ref[...]
    h = h_ref[...]
    gi = jnp.dot(x.astype(jnp.bfloat16), wih_ref[...].astype(jnp.bfloat16),
                 preferred_element_type=jnp.float32) + bih_ref[...]
    gh = jnp.dot(h.astype(jnp.bfloat16), whh_ref[...].astype(jnp.bfloat16),
                 preferred_element_type=jnp.float32) + bhh_ref[...]
    r = jax.nn.sigmoid(gi[:, :D] + gh[:, :D])
    z = jax.nn.sigmoid(gi[:, D:2 * D] + gh[:, D:2 * D])
    n = jnp.tanh(gi[:, 2 * D:] + r * gh[:, 2 * D:])
    out_ref[...] = (1.0 - z) * n + z * h


_GRU_BLK = 1024


def _gru(x, h, wih_t, whh_t, bih, bhh):
    grid = (B // _GRU_BLK,)
    return pl.pallas_call(
        _gru_block,
        grid=grid,
        in_specs=[
            pl.BlockSpec((_GRU_BLK, D), lambda i: (i, 0)),
            pl.BlockSpec((_GRU_BLK, D), lambda i: (i, 0)),
            pl.BlockSpec((D, 3 * D), lambda i: (0, 0)),
            pl.BlockSpec((D, 3 * D), lambda i: (0, 0)),
            pl.BlockSpec((1, 3 * D), lambda i: (0, 0)),
            pl.BlockSpec((1, 3 * D), lambda i: (0, 0)),
        ],
        out_specs=pl.BlockSpec((_GRU_BLK, D), lambda i: (i, 0)),
        out_shape=jax.ShapeDtypeStruct((B, D), jnp.float32),
    )(x, h, wih_t, whh_t, bih, bhh)


@functools.partial(
    pl.kernel,
    out_type=(
        jax.ShapeDtypeStruct((M, D), jnp.float32),
        jax.ShapeDtypeStruct((M,), jnp.float32),
    ),
    mesh=_mesh,
    scratch_types=[
        pltpu.VMEM((B,), jnp.int32),
        pltpu.VMEM((SH,), jnp.int32),
        pltpu.VMEM((SELR, IC), jnp.int32),
        pltpu.VMEM((SELR, IC), jnp.int32),
        pltpu.VMEM((IC, D), jnp.float32),
        pltpu.VMEM((IC, D), jnp.float32),
        pltpu.VMEM((B,), jnp.float32),
        pltpu.VMEM((SH,), jnp.float32),
        pltpu.VMEM((CP, D), jnp.float32),
        pltpu.VMEM((CP, D), jnp.float32),
        pltpu.SemaphoreType.DMA,
        pltpu.SemaphoreType.DMA,
        pltpu.SemaphoreType.DMA,
        pltpu.SemaphoreType.DMA,
        pltpu.SemaphoreType.DMA,
    ],
    compiler_params=_sc_params,
)
def _sc_scatter(mem_hbm, lu_hbm, ids_hbm, ts_hbm, upd_hbm,
                out_mem, out_lu, ids_v, win_v, selrow_v, selpos_v,
                rows0_v, rows1_v, ts_v, lu_v, cp0_v, cp1_v,
                sem_i, sem_o, sem_l, sem_g, sem_s):
    wid = lax.axis_index("s") * 2 + lax.axis_index("c")
    lo = wid * SH
    last = wid == NW - 1

    cbufs = (cp0_v, cp1_v)
    pltpu.sync_copy(ids_hbm, ids_v)
    pltpu.sync_copy(ts_hbm, ts_v)

    hi = jnp.where(last, M, lo + SH)
    neg1 = jnp.full((16,), -1, jnp.int32)
    for k in range(NV):
        win_v[pl.ds(k * 16, 16)] = neg1

    lane = lax.iota(jnp.int32, 16)

    SUF = 8

    def _scan(g, carry):
        for u in range(SUF):
            idv = ids_v[pl.ds(g * (16 * SUF) + u * 16, 16)]
            m = jnp.logical_and(idv >= lo, idv < hi)
            posv = g * (16 * SUF) + u * 16 + lane
            plsc.store_scatter(win_v, [idv - lo], posv, mask=m)
        return carry

    lax.fori_loop(0, B // (16 * SUF), _scan, 0)

    def _copy_shard(nfull, tail):
        sizes = [CP] * nfull + ([tail] if tail else [])
        nch = len(sizes)

        def rd(c):
            b = cbufs[c % 2].at[pl.ds(0, sizes[c])]
            return pltpu.make_async_copy(mem_hbm.at[pl.ds(lo + c * CP, sizes[c])], b, sem_i)

        def wr(c):
            b = cbufs[c % 2].at[pl.ds(0, sizes[c])]
            return pltpu.make_async_copy(b, out_mem.at[pl.ds(lo + c * CP, sizes[c])], sem_o)

        rd(0).start()
        for c in range(nch):
            rd(c).wait()
            wr(c).start()
            if c + 1 < nch:
                if c >= 1:
                    wr(c - 1).wait()
                rd(c + 1).start()
        wr(nch - 1).wait()
        if nch >= 2:
            wr(nch - 2).wait()


    def _compact(k, cnt):
        wv = win_v[pl.ds(k * 16, 16)]
        m = wv >= 0
        offs = cnt + plsc.cumsum(m.astype(jnp.int32)) - 1
        rowv = lo + k * 16 + lane
        plsc.store_scatter(selrow_v, [offs >> 7, offs & 127], rowv, mask=m)
        plsc.store_scatter(selpos_v, [offs >> 7, offs & 127], wv, mask=m)
        return cnt + jnp.sum(m.astype(jnp.int32))

    cnt = lax.fori_loop(0, NV, _compact, jnp.int32(0))

    @pl.when(cnt > 0)
    def _():
        row0 = jnp.full((16,), selrow_v[0, pl.ds(0, 16)][0], jnp.int32)
        pos0 = jnp.full((16,), selpos_v[0, pl.ds(0, 16)][0], jnp.int32)
        padded = ((cnt + IC - 1) >> 7) << 7
        for t in range(8):
            offs = cnt + t * 16 + lane
            mp = offs < padded
            plsc.store_scatter(selrow_v, [offs >> 7, offs & 127], row0, mask=mp)
            plsc.store_scatter(selpos_v, [offs >> 7, offs & 127], pos0, mask=mp)

    nch = (cnt + IC - 1) >> 7
    rbufs = (rows0_v, rows1_v)
    GB = 2

    def g_rows(ch):
        return pltpu.make_async_copy(upd_hbm.at[selpos_v.at[ch]], rbufs[ch % GB], sem_g)

    def s_rows(ch):
        return pltpu.make_async_copy(rbufs[ch % GB], out_mem.at[selrow_v.at[ch]], sem_s)

    for ch in range(GB):
        @pl.when(ch < nch)
        def _():
            g_rows(ch).start()

    @pl.when(jnp.logical_not(last))
    def _():
        pltpu.async_copy(lu_hbm.at[pl.ds(lo, SH)], lu_v, sem_l)
        _copy_shard(NCP, 0)

    @pl.when(last)
    def _():
        pltpu.async_copy(lu_hbm.at[pl.ds(lo, SH_LAST)], lu_v.at[pl.ds(0, SH_LAST)], sem_l)
        _copy_shard(NCP_LAST, CP_TAIL)

    @pl.when(jnp.logical_not(last))
    def _():
        pltpu.make_async_copy(lu_hbm.at[pl.ds(lo, SH)], lu_v, sem_l).wait()

    @pl.when(last)
    def _():
        pltpu.make_async_copy(lu_hbm.at[pl.ds(lo, SH_LAST)], lu_v.at[pl.ds(0, SH_LAST)], sem_l).wait()

    @pl.when(cnt > 0)
    def _():
        for q in range(SELR * (IC // 16)):
            base = q * 16
            mq = (base + lane) < cnt
            rv = selrow_v[q // 8, pl.ds((q % 8) * 16, 16)] - lo
            pv = selpos_v[q // 8, pl.ds((q % 8) * 16, 16)]
            tsv = plsc.load_gather(ts_v, [pv], mask=mq)
            plsc.store_scatter(lu_v, [rv], tsv, mask=mq)

        for r in range(0, SELR, GB):
            for ch in range(r, min(r + GB, SELR)):
                if r > 0:
                    @pl.when(ch < nch)
                    def _():
                        g_rows(ch).start()
            for ch in range(r, min(r + GB, SELR)):
                @pl.when(ch < nch)
                def _():
                    g_rows(ch).wait()
                    s_rows(ch).start()
            for ch in range(r, min(r + GB, SELR)):
                @pl.when(ch < nch)
                def _():
                    s_rows(ch).wait()

    @pl.when(jnp.logical_not(last))
    def _():
        pltpu.sync_copy(lu_v, out_lu.at[pl.ds(lo, SH)])

    @pl.when(last)
    def _():
        pltpu.sync_copy(lu_v.at[pl.ds(0, SH_LAST)], out_lu.at[pl.ds(lo, SH_LAST)])


def kernel(memory, last_update, unique_node_ids, unique_messages, timestamps, W_ih, W_hh, b_ih, b_hh):
    h = _sc_gather(memory, unique_node_ids)
    updated = _gru(unique_messages, h, W_ih.T, W_hh.T,
                   b_ih.reshape(1, 3 * D), b_hh.reshape(1, 3 * D))
    out_mem, out_lu = _sc_scatter(memory, last_update, unique_node_ids,
                                  timestamps, updated)
    return (out_mem, out_lu)


# speedup vs baseline: 1.0187x; 1.0187x over previous
import functools

import jax
import jax.numpy as jnp
from jax import lax
from jax.experimental import pallas as pl
from jax.experimental.pallas import tpu as pltpu
from jax.experimental.pallas import tpu_sc as plsc

M = 100000
D = 128
B = 16384
NW = 32
BPW = B // NW
IC = 128
NB_G = BPW // IC
SH = 3136
SH_LAST = M - (NW - 1) * SH
NV = SH // 16
SELR = SH // IC + 1
CP = 112
NCP = SH // CP
NCP_LAST = SH_LAST // CP
CP_TAIL = SH_LAST - NCP_LAST * CP

_mesh = plsc.VectorSubcoreMesh(core_axis_name="c", subcore_axis_name="s")
_sc_params = pltpu.CompilerParams(needs_layout_passes=False)


@functools.partial(
    pl.kernel,
    out_type=jax.ShapeDtypeStruct((B, D), jnp.float32),
    mesh=_mesh,
    scratch_types=[
        pltpu.VMEM((BPW,), jnp.int32),
        pltpu.VMEM((BPW, D), jnp.float32),
        pltpu.SemaphoreType.DMA,
    ],
    compiler_params=_sc_params,
)
def _sc_gather(mem_hbm, ids_hbm, h_hbm, idx_v, rows_v, sem):
    wid = lax.axis_index("s") * 2 + lax.axis_index("c")
    base = wid * BPW
    pltpu.sync_copy(ids_hbm.at[pl.ds(base, BPW)], idx_v)
    for k in range(NB_G):
        pltpu.async_copy(mem_hbm.at[idx_v.at[pl.ds(k * IC, IC)]],
                         rows_v.at[pl.ds(k * IC, IC)], sem)
    for k in range(NB_G):
        pltpu.make_async_copy(mem_hbm.at[idx_v.at[pl.ds(k * IC, IC)]],
                              rows_v.at[pl.ds(k * IC, IC)], sem).wait()
    pltpu.sync_copy(rows_v, h_hbm.at[pl.ds(base, BPW)])


def _gru_block(x_ref, h_ref, wih_ref, whh_ref, bih_ref, bhh_ref, out_ref):
    x = x_ref[...]
    h = h_ref[...]
    gi = jnp.dot(x.astype(jnp.bfloat16), wih_ref[...].astype(jnp.bfloat16),
                 preferred_element_type=jnp.float32) + bih_ref[...]
    gh = jnp.dot(h.astype(jnp.bfloat16), whh_ref[...].astype(jnp.bfloat16),
                 preferred_element_type=jnp.float32) + bhh_ref[...]
    r = jax.nn.sigmoid(gi[:, :D] + gh[:, :D])
    z = jax.nn.sigmoid(gi[:, D:2 * D] + gh[:, D:2 * D])
    n = jnp.tanh(gi[:, 2 * D:] + r * gh[:, 2 * D:])
    out_ref[...] = (1.0 - z) * n + z * h


_GRU_BLK = 1024


def _gru(x, h, wih_t, whh_t, bih, bhh):
    grid = (B // _GRU_BLK,)
    return pl.pallas_call(
        _gru_block,
        grid=grid,
        in_specs=[
            pl.BlockSpec((_GRU_BLK, D), lambda i: (i, 0)),
            pl.BlockSpec((_GRU_BLK, D), lambda i: (i, 0)),
            pl.BlockSpec((D, 3 * D), lambda i: (0, 0)),
            pl.BlockSpec((D, 3 * D), lambda i: (0, 0)),
            pl.BlockSpec((1, 3 * D), lambda i: (0, 0)),
            pl.BlockSpec((1, 3 * D), lambda i: (0, 0)),
        ],
        out_specs=pl.BlockSpec((_GRU_BLK, D), lambda i: (i, 0)),
        out_shape=jax.ShapeDtypeStruct((B, D), jnp.float32),
    )(x, h, wih_t, whh_t, bih, bhh)


@functools.partial(
    pl.kernel,
    out_type=(
        jax.ShapeDtypeStruct((M, D), jnp.float32),
        jax.ShapeDtypeStruct((M,), jnp.float32),
    ),
    mesh=_mesh,
    scratch_types=[
        pltpu.VMEM((B,), jnp.int32),
        pltpu.VMEM((SH,), jnp.int32),
        pltpu.VMEM((SELR, IC), jnp.int32),
        pltpu.VMEM((SELR, IC), jnp.int32),
        pltpu.VMEM((IC, D), jnp.float32),
        pltpu.VMEM((IC, D), jnp.float32),
        pltpu.VMEM((B,), jnp.float32),
        pltpu.VMEM((SH,), jnp.float32),
        pltpu.VMEM((CP, D), jnp.float32),
        pltpu.VMEM((CP, D), jnp.float32),
        pltpu.SemaphoreType.DMA,
        pltpu.SemaphoreType.DMA,
        pltpu.SemaphoreType.DMA,
        pltpu.SemaphoreType.DMA,
        pltpu.SemaphoreType.DMA,
    ],
    compiler_params=_sc_params,
)
def _sc_scatter(mem_hbm, lu_hbm, ids_hbm, ts_hbm, upd_hbm,
                out_mem, out_lu, ids_v, win_v, selrow_v, selpos_v,
                rows0_v, rows1_v, ts_v, lu_v, cp0_v, cp1_v,
                sem_i, sem_o, sem_l, sem_g, sem_s):
    wid = lax.axis_index("s") * 2 + lax.axis_index("c")
    lo = wid * SH
    last = wid == NW - 1

    cbufs = (cp0_v, cp1_v)
    pltpu.sync_copy(ids_hbm, ids_v)
    pltpu.sync_copy(ts_hbm, ts_v)

    hi = jnp.where(last, M, lo + SH)
    neg1 = jnp.full((16,), -1, jnp.int32)
    for k in range(NV):
        win_v[pl.ds(k * 16, 16)] = neg1

    lane = lax.iota(jnp.int32, 16)

    SUF = 8

    def _scan(g, carry):
        for u in range(SUF):
            idv = ids_v[pl.ds(g * (16 * SUF) + u * 16, 16)]
            m = jnp.logical_and(idv >= lo, idv < hi)
            posv = g * (16 * SUF) + u * 16 + lane
            plsc.store_scatter(win_v, [idv - lo], posv, mask=m)
        return carry

    def _copy_shard(nfull, tail):
        sizes = [CP] * nfull + ([tail] if tail else [])
        nch = len(sizes)

        def rd(c):
            b = cbufs[c % 2].at[pl.ds(0, sizes[c])]
            return pltpu.make_async_copy(mem_hbm.at[pl.ds(lo + c * CP, sizes[c])], b, sem_i)

        def wr(c):
            b = cbufs[c % 2].at[pl.ds(0, sizes[c])]
            return pltpu.make_async_copy(b, out_mem.at[pl.ds(lo + c * CP, sizes[c])], sem_o)

        rd(0).start()
        for c in range(nch):
            rd(c).wait()
            wr(c).start()
            if c + 1 < nch:
                if c >= 1:
                    wr(c - 1).wait()
                rd(c + 1).start()
            lax.fori_loop((c * (B // (16 * SUF))) // nch,
                          ((c + 1) * (B // (16 * SUF))) // nch, _scan, 0)
        wr(nch - 1).wait()
        if nch >= 2:
            wr(nch - 2).wait()

    @pl.when(jnp.logical_not(last))
    def _():
        pltpu.async_copy(lu_hbm.at[pl.ds(lo, SH)], lu_v, sem_l)
        _copy_shard(NCP, 0)

    @pl.when(last)
    def _():
        pltpu.async_copy(lu_hbm.at[pl.ds(lo, SH_LAST)], lu_v.at[pl.ds(0, SH_LAST)], sem_l)
        _copy_shard(NCP_LAST, CP_TAIL)

    def _compact(k, cnt):
        wv = win_v[pl.ds(k * 16, 16)]
        m = wv >= 0
        offs = cnt + plsc.cumsum(m.astype(jnp.int32)) - 1
        rowv = lo + k * 16 + lane
        plsc.store_scatter(selrow_v, [offs >> 7, offs & 127], rowv, mask=m)
        plsc.store_scatter(selpos_v, [offs >> 7, offs & 127], wv, mask=m)
        return cnt + jnp.sum(m.astype(jnp.int32))

    cnt = lax.fori_loop(0, NV, _compact, jnp.int32(0))

    @pl.when(jnp.logical_not(last))
    def _():
        pltpu.make_async_copy(lu_hbm.at[pl.ds(lo, SH)], lu_v, sem_l).wait()

    @pl.when(last)
    def _():
        pltpu.make_async_copy(lu_hbm.at[pl.ds(lo, SH_LAST)], lu_v.at[pl.ds(0, SH_LAST)], sem_l).wait()

    @pl.when(cnt > 0)
    def _():
        row0 = jnp.full((16,), selrow_v[0, pl.ds(0, 16)][0], jnp.int32)
        pos0 = jnp.full((16,), selpos_v[0, pl.ds(0, 16)][0], jnp.int32)
        nch = (cnt + IC - 1) >> 7
        padded = nch << 7
        for t in range(8):
            offs = cnt + t * 16 + lane
            mp = offs < padded
            plsc.store_scatter(selrow_v, [offs >> 7, offs & 127], row0, mask=mp)
            plsc.store_scatter(selpos_v, [offs >> 7, offs & 127], pos0, mask=mp)

        for q in range(SELR * (IC // 16)):
            base = q * 16
            mq = (base + lane) < cnt
            rv = selrow_v[q // 8, pl.ds((q % 8) * 16, 16)] - lo
            pv = selpos_v[q // 8, pl.ds((q % 8) * 16, 16)]
            tsv = plsc.load_gather(ts_v, [pv], mask=mq)
            plsc.store_scatter(lu_v, [rv], tsv, mask=mq)

        rbufs = (rows0_v, rows1_v)
        GB = 2

        def g_rows(ch):
            return pltpu.make_async_copy(upd_hbm.at[selpos_v.at[ch]], rbufs[ch % GB], sem_g)

        def s_rows(ch):
            return pltpu.make_async_copy(rbufs[ch % GB], out_mem.at[selrow_v.at[ch]], sem_s)

        for r in range(0, SELR, GB):
            for ch in range(r, min(r + GB, SELR)):
                @pl.when(ch < nch)
                def _():
                    g_rows(ch).start()
            for ch in range(r, min(r + GB, SELR)):
                @pl.when(ch < nch)
                def _():
                    g_rows(ch).wait()
                    s_rows(ch).start()
            for ch in range(r, min(r + GB, SELR)):
                @pl.when(ch < nch)
                def _():
                    s_rows(ch).wait()

    @pl.when(jnp.logical_not(last))
    def _():
        pltpu.sync_copy(lu_v, out_lu.at[pl.ds(lo, SH)])

    @pl.when(last)
    def _():
        pltpu.sync_copy(lu_v.at[pl.ds(0, SH_LAST)], out_lu.at[pl.ds(lo, SH_LAST)])


def kernel(memory, last_update, unique_node_ids, unique_messages, timestamps, W_ih, W_hh, b_ih, b_hh):
    h = _sc_gather(memory, unique_node_ids)
    updated = _gru(unique_messages, h, W_ih.T, W_hh.T,
                   b_ih.reshape(1, 3 * D), b_hh.reshape(1, 3 * D))
    out_mem, out_lu = _sc_scatter(memory, last_update, unique_node_ids,
                                  timestamps, updated)
    return (out_mem, out_lu)
